# baseline (speedup 1.0000x reference)
import functools

import jax
import jax.numpy as jnp
import numpy as np
from jax import lax
from jax.experimental import pallas as pl
from jax.experimental.pallas import tpu as pltpu
from jax.experimental.pallas import tpu_sc as plsc

N_NODES = 10000
N_EDGES = 160000
D_FEAT = 128
D_EDGE = 16
H = 16
N_PRED = 100000


def _nodeproj_body(x_ref, w_ref, b_ref, o_ref):
    o_ref[...] = (
        jnp.dot(x_ref[...], w_ref[...], preferred_element_type=jnp.float32)
        + b_ref[...]
    )


def _node_projection(x, w, b):
    blk = 1000
    return pl.pallas_call(
        _nodeproj_body,
        grid=(N_NODES // blk,),
        in_specs=[
            pl.BlockSpec((blk, D_FEAT), lambda i: (i, 0)),
            pl.BlockSpec((D_FEAT, H), lambda i: (0, 0)),
            pl.BlockSpec((1, H), lambda i: (0, 0)),
        ],
        out_specs=pl.BlockSpec((blk, H), lambda i: (i, 0)),
        out_shape=jax.ShapeDtypeStruct((N_NODES, H), jnp.float32),
    )(x, w, b.reshape(1, H))


def _msg_body(ef_ref, hs_ref, wen_ref, ben_ref, r_ref, s_ref, o_ref):
    we = jnp.maximum(
        jnp.dot(ef_ref[...], wen_ref[...], preferred_element_type=jnp.float32)
        + ben_ref[...],
        0.0,
    )
    rep = jnp.dot(hs_ref[...], r_ref[...], preferred_element_type=jnp.float32)
    o_ref[...] = jnp.dot(rep * we, s_ref[...], preferred_element_type=jnp.float32)


_EBLK = 1280


def _edge_messages(ef, hs, wen, ben, rmat, smat):
    return pl.pallas_call(
        _msg_body,
        grid=(N_EDGES // _EBLK,),
        in_specs=[
            pl.BlockSpec((_EBLK, D_EDGE), lambda i: (i, 0)),
            pl.BlockSpec((_EBLK, H), lambda i: (i, 0)),
            pl.BlockSpec((D_EDGE, H * H), lambda i: (0, 0)),
            pl.BlockSpec((1, H * H), lambda i: (0, 0)),
            pl.BlockSpec((H, H * H), lambda i: (0, 0)),
            pl.BlockSpec((H * H, H), lambda i: (0, 0)),
        ],
        out_specs=pl.BlockSpec((_EBLK, H), lambda i: (i, 0)),
        out_shape=jax.ShapeDtypeStruct((N_EDGES, H), jnp.float32),
    )(ef, hs, wen, ben, rmat, smat)


def _pred_body(a_ref, b_ref, w1a_ref, w1b_ref, bp1_ref, w2_ref, bp2_ref, o_ref):
    z = jnp.maximum(
        jnp.dot(a_ref[...], w1a_ref[...], preferred_element_type=jnp.float32)
        + jnp.dot(b_ref[...], w1b_ref[...], preferred_element_type=jnp.float32)
        + bp1_ref[...],
        0.0,
    )
    o_ref[...] = jnp.sum(z * w2_ref[...], axis=1, keepdims=True) + bp2_ref[...]


_PBLK = 1280
_NPAD = 102400


def _edge_predictor(pair_h, w1a, w1b, bp1, w2row, bp2):
    nblk = _NPAD // _PBLK
    return pl.pallas_call(
        _pred_body,
        grid=(nblk,),
        in_specs=[
            pl.BlockSpec((_PBLK, H), lambda i: (i, 0)),
            pl.BlockSpec((_PBLK, H), lambda i, n=nblk: (i + n, 0)),
            pl.BlockSpec((H, H), lambda i: (0, 0)),
            pl.BlockSpec((H, H), lambda i: (0, 0)),
            pl.BlockSpec((1, H), lambda i: (0, 0)),
            pl.BlockSpec((1, H), lambda i: (0, 0)),
            pl.BlockSpec((1, 1), lambda i: (0, 0)),
        ],
        out_specs=pl.BlockSpec((_PBLK, 1), lambda i: (i, 0)),
        out_shape=jax.ShapeDtypeStruct((_NPAD, 1), jnp.float32),
    )(pair_h, pair_h, w1a, w1b, bp1, w2row, bp2)


def _gather_rows(table, idx):
    return table[idx]


def _segment_mean_relu(m, dst, bias):
    s = jax.ops.segment_sum(m, dst, num_segments=N_NODES)
    cnt = jax.ops.segment_sum(jnp.ones((m.shape[0],), jnp.float32), dst,
                              num_segments=N_NODES)
    return jnp.maximum(s / jnp.maximum(cnt, 1.0)[:, None] + bias, 0.0)


_R = jnp.asarray(np.kron(np.eye(H), np.ones((1, H))), dtype=jnp.float32)
_S = jnp.asarray(np.kron(np.ones((H, 1)), np.eye(H)), dtype=jnp.float32)


def kernel(x, edge_index, edge_feat, edge_list, W_np, b_np, W_en, b_en,
           b1, b2, W_p1, b_p1, W_p2, b_p2):
    src = edge_index[0]
    dst = edge_index[1]

    h = _node_projection(x, W_np, b_np)

    for bias in (b1, b2):
        hs = _gather_rows(h, src)
        m = _edge_messages(edge_feat, hs, W_en, b_en.reshape(1, H * H), _R, _S)
        h = _segment_mean_relu(m, dst, bias)

    pad = jnp.zeros((_NPAD - N_PRED,), jnp.int32)
    idx_all = jnp.concatenate(
        [edge_list[:, 0], pad, edge_list[:, 1], pad])
    pair_h = _gather_rows(h, idx_all)

    logits = _edge_predictor(
        pair_h, W_p1[:H], W_p1[H:], b_p1.reshape(1, H),
        W_p2.reshape(1, H), b_p2.reshape(1, 1))
    return logits[:N_PRED]


# speedup vs baseline: 1.5236x; 1.5236x over previous
import functools

import jax
import jax.numpy as jnp
import numpy as np
from jax import lax
from jax.experimental import pallas as pl
from jax.experimental.pallas import tpu as pltpu
from jax.experimental.pallas import tpu_sc as plsc

N_NODES = 10000
N_EDGES = 160000
D_FEAT = 128
D_EDGE = 16
H = 16
N_PRED = 100000


def _nodeproj_body(x_ref, w_ref, b_ref, o_ref):
    o_ref[...] = (
        jnp.dot(x_ref[...], w_ref[...], preferred_element_type=jnp.float32)
        + b_ref[...]
    )


def _node_projection(x, w, b):
    blk = 1000
    return pl.pallas_call(
        _nodeproj_body,
        grid=(N_NODES // blk,),
        in_specs=[
            pl.BlockSpec((blk, D_FEAT), lambda i: (i, 0)),
            pl.BlockSpec((D_FEAT, H), lambda i: (0, 0)),
            pl.BlockSpec((1, H), lambda i: (0, 0)),
        ],
        out_specs=pl.BlockSpec((blk, H), lambda i: (i, 0)),
        out_shape=jax.ShapeDtypeStruct((N_NODES, H), jnp.float32),
    )(x, w, b.reshape(1, H))


def _msg_body(ef_ref, hs_ref, wen_ref, ben_ref, r_ref, s_ref, o_ref):
    we = jnp.maximum(
        jnp.dot(ef_ref[...], wen_ref[...], preferred_element_type=jnp.float32)
        + ben_ref[...],
        0.0,
    )
    rep = jnp.dot(hs_ref[...], r_ref[...], preferred_element_type=jnp.float32)
    o_ref[...] = jnp.dot(rep * we, s_ref[...], preferred_element_type=jnp.float32)


_EBLK = 1280
_EPAD = 163840


def _edge_messages(ef, hs, wen, ben, rmat, smat):
    return pl.pallas_call(
        _msg_body,
        grid=(N_EDGES // _EBLK,),
        in_specs=[
            pl.BlockSpec((_EBLK, D_EDGE), lambda i: (i, 0)),
            pl.BlockSpec((_EBLK, H), lambda i: (i, 0)),
            pl.BlockSpec((D_EDGE, H * H), lambda i: (0, 0)),
            pl.BlockSpec((1, H * H), lambda i: (0, 0)),
            pl.BlockSpec((H, H * H), lambda i: (0, 0)),
            pl.BlockSpec((H * H, H), lambda i: (0, 0)),
        ],
        out_specs=pl.BlockSpec((_EBLK, H), lambda i: (i, 0)),
        out_shape=jax.ShapeDtypeStruct((_EPAD, H), jnp.float32),
    )(ef, hs, wen, ben, rmat, smat)


def _pred_body(a_ref, b_ref, w1a_ref, w1b_ref, bp1_ref, w2_ref, bp2_ref, o_ref):
    z = jnp.maximum(
        jnp.dot(a_ref[...], w1a_ref[...], preferred_element_type=jnp.float32)
        + jnp.dot(b_ref[...], w1b_ref[...], preferred_element_type=jnp.float32)
        + bp1_ref[...],
        0.0,
    )
    o_ref[...] = jnp.sum(z * w2_ref[...], axis=1, keepdims=True) + bp2_ref[...]


_PBLK = 1280
_NPAD = 102400


def _edge_predictor(pair_h, w1a, w1b, bp1, w2row, bp2):
    nblk = _NPAD // _PBLK
    return pl.pallas_call(
        _pred_body,
        grid=(nblk,),
        in_specs=[
            pl.BlockSpec((_PBLK, H), lambda i: (i, 0)),
            pl.BlockSpec((_PBLK, H), lambda i, n=nblk: (i + n, 0)),
            pl.BlockSpec((H, H), lambda i: (0, 0)),
            pl.BlockSpec((H, H), lambda i: (0, 0)),
            pl.BlockSpec((1, H), lambda i: (0, 0)),
            pl.BlockSpec((1, H), lambda i: (0, 0)),
            pl.BlockSpec((1, 1), lambda i: (0, 0)),
        ],
        out_specs=pl.BlockSpec((_PBLK, 1), lambda i: (i, 0)),
        out_shape=jax.ShapeDtypeStruct((_NPAD, 1), jnp.float32),
    )(pair_h, pair_h, w1a, w1b, bp1, w2row, bp2)


_NW = 32


def _make_sc_gather(n_idx):
    n_rows128 = n_idx // 128
    ro = 8
    n_outer = n_rows128 // ro
    iters = -(-n_outer // _NW)
    ch = ro * 128

    @functools.partial(
        pl.kernel,
        out_type=jax.ShapeDtypeStruct((n_idx, H), jnp.float32),
        mesh=plsc.VectorSubcoreMesh(core_axis_name="c", subcore_axis_name="s"),
        compiler_params=pltpu.CompilerParams(use_tc_tiling_on_sc=False),
        scratch_types=[
            pltpu.VMEM((ro, 128), jnp.int32),
            pltpu.VMEM((ch, H), jnp.float32),
            pltpu.SemaphoreType.DMA,
        ],
    )
    def gk(table_hbm, idx_hbm, out_hbm, idx_v, rows_v, sem):
        wid = lax.axis_index("s") * 2 + lax.axis_index("c")

        def body(t, carry):
            cid = t * _NW + wid

            @pl.when(cid < n_outer)
            def _():
                pltpu.sync_copy(idx_hbm.at[pl.ds(cid * ro, ro)], idx_v)
                descs = [
                    pltpu.async_copy(
                        table_hbm.at[idx_v.at[j]],
                        rows_v.at[pl.ds(j * 128, 128)], sem)
                    for j in range(ro)
                ]
                for d in descs:
                    d.wait()
                pltpu.sync_copy(rows_v, out_hbm.at[pl.ds(cid * ch, ch)])

            return carry

        lax.fori_loop(0, iters, body, 0)

    return gk


def _segment_mean_relu(m, dst, bias):
    s = jax.ops.segment_sum(m, dst, num_segments=N_NODES)
    cnt = jax.ops.segment_sum(jnp.ones((m.shape[0],), jnp.float32), dst,
                              num_segments=N_NODES)
    return jnp.maximum(s / jnp.maximum(cnt, 1.0)[:, None] + bias, 0.0)


_R = jnp.asarray(np.kron(np.eye(H), np.ones((1, H))), dtype=jnp.float32)
_S = jnp.asarray(np.kron(np.ones((H, 1)), np.eye(H)), dtype=jnp.float32)


def kernel(x, edge_index, edge_feat, edge_list, W_np, b_np, W_en, b_en,
           b1, b2, W_p1, b_p1, W_p2, b_p2):
    epad = jnp.zeros((_EPAD - N_EDGES,), jnp.int32)
    src2d = jnp.concatenate([edge_index[0], epad]).reshape(_EPAD // 128, 128)
    dst = edge_index[1]

    h = _node_projection(x, W_np, b_np)

    conv_gather = _make_sc_gather(_EPAD)
    for bias in (b1, b2):
        hs = conv_gather(h, src2d)
        m = _edge_messages(edge_feat, hs, W_en, b_en.reshape(1, H * H), _R, _S)
        h = _segment_mean_relu(m[:N_EDGES], dst, bias)

    pad = jnp.zeros((_NPAD - N_PRED,), jnp.int32)
    idx_all = jnp.concatenate(
        [edge_list[:, 0], pad, edge_list[:, 1], pad]).reshape(
            2 * _NPAD // 128, 128)
    pair_h = _make_sc_gather(2 * _NPAD)(h, idx_all)

    logits = _edge_predictor(
        pair_h, W_p1[:H], W_p1[H:], b_p1.reshape(1, H),
        W_p2.reshape(1, H), b_p2.reshape(1, 1))
    return logits[:N_PRED]


# speedup vs baseline: 2.4648x; 1.6177x over previous
import functools

import jax
import jax.numpy as jnp
import numpy as np
from jax import lax
from jax.experimental import pallas as pl
from jax.experimental.pallas import tpu as pltpu
from jax.experimental.pallas import tpu_sc as plsc

N_NODES = 10000
N_EDGES = 160000
D_FEAT = 128
D_EDGE = 16
H = 16
N_PRED = 100000


def _nodeproj_body(x_ref, w_ref, b_ref, o_ref):
    o_ref[...] = (
        jnp.dot(x_ref[...], w_ref[...], preferred_element_type=jnp.float32)
        + b_ref[...]
    )


def _node_projection(x, w, b):
    blk = 1000
    return pl.pallas_call(
        _nodeproj_body,
        grid=(N_NODES // blk,),
        in_specs=[
            pl.BlockSpec((blk, D_FEAT), lambda i: (i, 0)),
            pl.BlockSpec((D_FEAT, H), lambda i: (0, 0)),
            pl.BlockSpec((1, H), lambda i: (0, 0)),
        ],
        out_specs=pl.BlockSpec((blk, H), lambda i: (i, 0)),
        out_shape=jax.ShapeDtypeStruct((N_NODES, H), jnp.float32),
    )(x, w, b.reshape(1, H))


def _msg_body(ef_ref, hs_ref, wen_ref, ben_ref, r_ref, s_ref, o_ref):
    we = jnp.maximum(
        jnp.dot(ef_ref[...], wen_ref[...], preferred_element_type=jnp.float32)
        + ben_ref[...],
        0.0,
    )
    rep = jnp.dot(hs_ref[...], r_ref[...], preferred_element_type=jnp.float32)
    o_ref[...] = jnp.dot(rep * we, s_ref[...], preferred_element_type=jnp.float32)


_EBLK = 1280
_EPAD = 163840


def _edge_messages(ef, hs, wen, ben, rmat, smat):
    return pl.pallas_call(
        _msg_body,
        grid=(N_EDGES // _EBLK,),
        in_specs=[
            pl.BlockSpec((_EBLK, D_EDGE), lambda i: (i, 0)),
            pl.BlockSpec((_EBLK, H), lambda i: (i, 0)),
            pl.BlockSpec((D_EDGE, H * H), lambda i: (0, 0)),
            pl.BlockSpec((1, H * H), lambda i: (0, 0)),
            pl.BlockSpec((H, H * H), lambda i: (0, 0)),
            pl.BlockSpec((H * H, H), lambda i: (0, 0)),
        ],
        out_specs=pl.BlockSpec((_EBLK, H), lambda i: (i, 0)),
        out_shape=jax.ShapeDtypeStruct((_EPAD, H), jnp.float32),
    )(ef, hs, wen, ben, rmat, smat)


def _pred_body(a_ref, b_ref, w1a_ref, w1b_ref, bp1_ref, w2_ref, bp2_ref, o_ref):
    z = jnp.maximum(
        jnp.dot(a_ref[...], w1a_ref[...], preferred_element_type=jnp.float32)
        + jnp.dot(b_ref[...], w1b_ref[...], preferred_element_type=jnp.float32)
        + bp1_ref[...],
        0.0,
    )
    o_ref[...] = jnp.sum(z * w2_ref[...], axis=1, keepdims=True) + bp2_ref[...]


_PBLK = 1280
_NPAD = 102400


def _edge_predictor(pair_h, w1a, w1b, bp1, w2row, bp2):
    nblk = _NPAD // _PBLK
    return pl.pallas_call(
        _pred_body,
        grid=(nblk,),
        in_specs=[
            pl.BlockSpec((_PBLK, H), lambda i: (i, 0)),
            pl.BlockSpec((_PBLK, H), lambda i, n=nblk: (i + n, 0)),
            pl.BlockSpec((H, H), lambda i: (0, 0)),
            pl.BlockSpec((H, H), lambda i: (0, 0)),
            pl.BlockSpec((1, H), lambda i: (0, 0)),
            pl.BlockSpec((1, H), lambda i: (0, 0)),
            pl.BlockSpec((1, 1), lambda i: (0, 0)),
        ],
        out_specs=pl.BlockSpec((_PBLK, 1), lambda i: (i, 0)),
        out_shape=jax.ShapeDtypeStruct((_NPAD, 1), jnp.float32),
    )(pair_h, pair_h, w1a, w1b, bp1, w2row, bp2)


_NW = 32


def _make_sc_gather(n_idx):
    n_rows128 = n_idx // 128
    ro = 8
    n_outer = n_rows128 // ro
    iters = -(-n_outer // _NW)
    ch = ro * 128

    @functools.partial(
        pl.kernel,
        out_type=jax.ShapeDtypeStruct((n_idx, H), jnp.float32),
        mesh=plsc.VectorSubcoreMesh(core_axis_name="c", subcore_axis_name="s"),
        compiler_params=pltpu.CompilerParams(use_tc_tiling_on_sc=False),
        scratch_types=[
            pltpu.VMEM((ro, 128), jnp.int32),
            pltpu.VMEM((ch, H), jnp.float32),
            pltpu.SemaphoreType.DMA,
        ],
    )
    def gk(table_hbm, idx_hbm, out_hbm, idx_v, rows_v, sem):
        wid = lax.axis_index("s") * 2 + lax.axis_index("c")

        def body(t, carry):
            cid = t * _NW + wid

            @pl.when(cid < n_outer)
            def _():
                pltpu.sync_copy(idx_hbm.at[pl.ds(cid * ro, ro)], idx_v)
                descs = [
                    pltpu.async_copy(
                        table_hbm.at[idx_v.at[j]],
                        rows_v.at[pl.ds(j * 128, 128)], sem)
                    for j in range(ro)
                ]
                for d in descs:
                    d.wait()
                pltpu.sync_copy(rows_v, out_hbm.at[pl.ds(cid * ch, ch)])

            return carry

        lax.fori_loop(0, iters, body, 0)

    return gk


_HALF = N_NODES // 2
_ACC = 5120
_RPW = _ACC // 16


def _make_sc_scatter_mean():
    n_outer = _EPAD // 1024
    iters = n_outer // 16

    @functools.partial(
        pl.kernel,
        out_type=jax.ShapeDtypeStruct((N_NODES, H), jnp.float32),
        mesh=plsc.VectorSubcoreMesh(core_axis_name="c", subcore_axis_name="s"),
        compiler_params=pltpu.CompilerParams(use_tc_tiling_on_sc=False),
        scratch_types=[
            pltpu.VMEM((1024,), jnp.int32),
            pltpu.VMEM((8, 128), jnp.int32),
            pltpu.VMEM((1024, H), jnp.float32),
            pltpu.VMEM((128, H), jnp.float32),
            pltpu.VMEM((_RPW, H), jnp.float32),
            pltpu.VMEM((_RPW, H), jnp.float32),
            pltpu.VMEM((_RPW, H), jnp.float32),
            pltpu.VMEM((H,), jnp.float32),
            pltpu.VMEM_SHARED((_ACC, H), jnp.float32),
            pltpu.VMEM_SHARED((_ACC, H), jnp.float32),
            pltpu.SemaphoreType.DMA,
        ],
    )
    def sk(m_hbm, dst_hbm, bias_hbm, out_hbm,
           dstv, lidx, mrows, onesv, accv, cntv, outv, biasv,
           acc_sh, cnt_sh, sem):
        c = lax.axis_index("c")
        s = lax.axis_index("s")
        lo = c * _HALF

        def zr(i, carry):
            accv[i, :] = jnp.zeros((H,), jnp.float32)
            return carry

        lax.fori_loop(0, _RPW, zr, 0)

        def on(i, carry):
            onesv[i, :] = jnp.full((H,), 1.0, jnp.float32)
            return carry

        lax.fori_loop(0, 128, on, 0)

        pltpu.sync_copy(accv, acc_sh.at[pl.ds(s * _RPW, _RPW)])
        pltpu.sync_copy(accv, cnt_sh.at[pl.ds(s * _RPW, _RPW)])
        plsc.subcore_barrier()

        def body(t, carry):
            cid = t * 16 + s
            base = cid * 1024
            pltpu.sync_copy(dst_hbm.at[pl.ds(base, 1024)], dstv)
            for j in range(8):
                for k in range(8):
                    v = dstv[pl.ds((j * 8 + k) * 16, 16)]
                    inr = (v >= lo) & (v < lo + _HALF)
                    lv = jnp.where(inr, v - lo, _HALF)
                    lidx[j, pl.ds(k * 16, 16)] = lv
            pltpu.sync_copy(m_hbm.at[pl.ds(base, 1024)], mrows)
            descs = []
            for j in range(8):
                descs.append(pltpu.async_copy(
                    mrows.at[pl.ds(j * 128, 128)],
                    acc_sh.at[lidx.at[j]], sem, add=True))
                descs.append(pltpu.async_copy(
                    onesv, cnt_sh.at[lidx.at[j]], sem, add=True))
            for d in descs:
                d.wait()
            return carry

        lax.fori_loop(0, iters, body, 0)
        plsc.subcore_barrier()

        pltpu.sync_copy(acc_sh.at[pl.ds(s * _RPW, _RPW)], accv)
        pltpu.sync_copy(cnt_sh.at[pl.ds(s * _RPW, _RPW)], cntv)
        pltpu.sync_copy(bias_hbm, biasv)
        bv = biasv[...]

        def nb(i, carry):
            a = accv[i, :]
            n = cntv[i, :]
            outv[i, :] = jnp.maximum(a / jnp.maximum(n, 1.0) + bv, 0.0)
            return carry

        lax.fori_loop(0, _RPW, nb, 0)

        @pl.when(s < 15)
        def _():
            pltpu.sync_copy(outv, out_hbm.at[pl.ds(lo + s * _RPW, _RPW)])

        @pl.when(s == 15)
        def _():
            rem = _HALF - 15 * _RPW
            pltpu.sync_copy(outv.at[pl.ds(0, rem)],
                            out_hbm.at[pl.ds(lo + 15 * _RPW, rem)])

    return sk


def _segment_mean_relu(m, dst, bias):
    s = jax.ops.segment_sum(m, dst, num_segments=N_NODES)
    cnt = jax.ops.segment_sum(jnp.ones((m.shape[0],), jnp.float32), dst,
                              num_segments=N_NODES)
    return jnp.maximum(s / jnp.maximum(cnt, 1.0)[:, None] + bias, 0.0)


_R = jnp.asarray(np.kron(np.eye(H), np.ones((1, H))), dtype=jnp.float32)
_S = jnp.asarray(np.kron(np.ones((H, 1)), np.eye(H)), dtype=jnp.float32)


def kernel(x, edge_index, edge_feat, edge_list, W_np, b_np, W_en, b_en,
           b1, b2, W_p1, b_p1, W_p2, b_p2):
    epad = jnp.zeros((_EPAD - N_EDGES,), jnp.int32)
    src2d = jnp.concatenate([edge_index[0], epad]).reshape(_EPAD // 128, 128)
    dst_pad = jnp.concatenate([edge_index[1], epad + N_NODES])

    h = _node_projection(x, W_np, b_np)

    conv_gather = _make_sc_gather(_EPAD)
    scatter_mean = _make_sc_scatter_mean()
    for bias in (b1, b2):
        hs = conv_gather(h, src2d)
        m = _edge_messages(edge_feat, hs, W_en, b_en.reshape(1, H * H), _R, _S)
        h = scatter_mean(m, dst_pad, bias)

    pad = jnp.zeros((_NPAD - N_PRED,), jnp.int32)
    idx_all = jnp.concatenate(
        [edge_list[:, 0], pad, edge_list[:, 1], pad]).reshape(
            2 * _NPAD // 128, 128)
    pair_h = _make_sc_gather(2 * _NPAD)(h, idx_all)

    logits = _edge_predictor(
        pair_h, W_p1[:H], W_p1[H:], b_p1.reshape(1, H),
        W_p2.reshape(1, H), b_p2.reshape(1, 1))
    return logits[:N_PRED]
